# baseline (speedup 1.0000x reference)
import functools

import jax
import jax.numpy as jnp
from jax.experimental import pallas as pl
from jax.experimental.pallas import tpu as pltpu
from jax.sharding import PartitionSpec as P

_K = 9
_DIL = 2
_TOPK = _K * _DIL
_LANES = 128
_ROWS = 256


def _knn_block(xr_ref, xa_ref, nn_ref, cen_ref, *, rows: int, n: int):
    i = pl.program_id(1)
    xr = xr_ref[0]
    xa = xa_ref[0]

    inner = jax.lax.dot_general(
        xr, xa, (((0,), (0,)), ((), ())),
        preferred_element_type=jnp.float32,
        precision=jax.lax.Precision.DEFAULT,
    )
    sq_r = jnp.sum(xr * xr, axis=0)[:, None]
    sq_a = jnp.sum(xa * xa, axis=0)[None, :]
    d = sq_r - 2.0 * inner + sq_a

    nchunks = n // _LANES
    colf = jax.lax.broadcasted_iota(jnp.int32, (rows, n), 1).astype(jnp.float32)
    chunks = [d[:, c * _LANES:(c + 1) * _LANES] for c in range(nchunks)]
    inf = jnp.float32(jnp.inf)

    out_lane = jax.lax.broadcasted_iota(jnp.int32, (rows, _K), 1)
    acc = jnp.zeros((rows, _K), dtype=jnp.int32)
    prev = jnp.full((rows, 1), -1.0, dtype=jnp.float32)
    for t in range(_TOPK - 1):
        bv = None
        bi = None
        for c in range(nchunks):
            cif = colf[:, c * _LANES:(c + 1) * _LANES]
            cv = jnp.where(cif == prev, inf, chunks[c])
            chunks[c] = cv
            if bv is None:
                bv, bi = cv, cif
            else:
                take = cv < bv
                bv = jnp.minimum(bv, cv)
                bi = jnp.where(take, cif, bi)
        m = jnp.min(bv, axis=1, keepdims=True)
        idxf = jnp.min(jnp.where(bv == m, bi, inf), axis=1, keepdims=True)
        prev = idxf
        if t % _DIL == 0:
            idx = idxf.astype(jnp.int32)
            acc = jnp.where(out_lane == t // _DIL, idx, acc)

    nn_ref[0] = acc
    cen_ref[0] = (i * rows
                  + jax.lax.broadcasted_iota(jnp.int32, (rows, _K), 0))


def _knn_call(xdn):
    b, dim, n = xdn.shape
    grid = (b, n // _ROWS)
    return pl.pallas_call(
        functools.partial(_knn_block, rows=_ROWS, n=n),
        grid=grid,
        in_specs=[
            pl.BlockSpec((1, dim, _ROWS), lambda bi, i: (bi, 0, i)),
            pl.BlockSpec((1, dim, n), lambda bi, i: (bi, 0, 0)),
        ],
        out_specs=[
            pl.BlockSpec((1, _ROWS, _K), lambda bi, i: (bi, i, 0)),
            pl.BlockSpec((1, _ROWS, _K), lambda bi, i: (bi, i, 0)),
        ],
        out_shape=[
            jax.ShapeDtypeStruct((b, n, _K), jnp.int32),
            jax.ShapeDtypeStruct((b, n, _K), jnp.int32),
        ],
        compiler_params=pltpu.CompilerParams(
            dimension_semantics=("parallel", "parallel")),
    )(xdn, xdn)


@jax.jit
def kernel(x):
    b, dim, n, _ = x.shape
    xdn = jnp.reshape(x, (b, dim, n))

    nd = jax.device_count()
    nd = 2 if (nd >= 2 and b % 2 == 0) else 1
    if nd > 1:
        mesh = jax.make_mesh((nd,), ("d",))
        xdn = jax.reshard(xdn, jax.NamedSharding(mesh, P("d")))
        nn, cen = jax.shard_map(
            _knn_call, mesh=mesh, in_specs=P("d"),
            out_specs=(P("d"), P("d")), check_vma=False)(xdn)
    else:
        nn, cen = _knn_call(xdn)
    return jnp.stack((nn, cen), axis=0)
